# baseline (speedup 1.0000x reference)
import functools

import jax
import jax.numpy as jnp
from jax import lax
from jax.experimental import pallas as pl
from jax.experimental.pallas import tpu as pltpu
from jax.experimental.pallas import tpu_sc as plsc

VOCAB = 1000000
DIM = 32
BATCH = 16384

_NUM_CORES = 2
_NUM_SUBCORES = 16
_NW = _NUM_CORES * _NUM_SUBCORES
_B_PER_W = BATCH // _NW
_CHUNK = 128
_NCHUNK = _B_PER_W // _CHUNK


def _build_gather():
    mesh = plsc.VectorSubcoreMesh(core_axis_name="c", subcore_axis_name="s")

    @functools.partial(
        pl.kernel,
        mesh=mesh,
        out_type=jax.ShapeDtypeStruct((BATCH, DIM), jnp.float32),
        scratch_types=[
            pltpu.VMEM((_NCHUNK, _CHUNK), jnp.int32),
            pltpu.VMEM((_NCHUNK, _CHUNK, DIM), jnp.float32),
            pltpu.SemaphoreType.DMA,
        ],
        compiler_params=pltpu.CompilerParams(use_tc_tiling_on_sc=False),
    )
    def gather_kernel(table_hbm, idx_hbm, out_hbm, idx_v, rows_v, sem):
        wid = lax.axis_index("s") * _NUM_CORES + lax.axis_index("c")
        pltpu.sync_copy(idx_hbm.at[pl.ds(wid * _NCHUNK, _NCHUNK)], idx_v)
        copies = []
        for j in range(_NCHUNK):
            copies.append(
                pltpu.async_copy(table_hbm.at[idx_v.at[j]], rows_v.at[j], sem)
            )
        for j in range(_NCHUNK):
            copies[j].wait()
            pltpu.sync_copy(
                rows_v.at[j],
                out_hbm.at[pl.ds(wid * _B_PER_W + j * _CHUNK, _CHUNK)],
            )

    return gather_kernel


_gather = _build_gather()


def kernel(table, feedid):
    return _gather(table, feedid.reshape(BATCH // _CHUNK, _CHUNK))


# speedup vs baseline: 3.5286x; 3.5286x over previous
import functools

import jax
import jax.numpy as jnp
from jax import lax
from jax.experimental import pallas as pl
from jax.experimental.pallas import tpu as pltpu
from jax.experimental.pallas import tpu_sc as plsc

VOCAB = 1000000
DIM = 32
BATCH = 16384

_NUM_CORES = 2
_NUM_SUBCORES = 16
_NW = _NUM_CORES * _NUM_SUBCORES
_B_PER_W = BATCH // _NW
_GRP = 8
_NITER = _B_PER_W // (2 * _GRP)


def _build_gather():
    mesh = plsc.VectorSubcoreMesh(core_axis_name="c", subcore_axis_name="s")

    @functools.partial(
        pl.kernel,
        mesh=mesh,
        out_type=jax.ShapeDtypeStruct((BATCH * DIM,), jnp.float32),
        scratch_types=[
            pltpu.VMEM((_B_PER_W,), jnp.int32),
            pltpu.VMEM((_GRP, DIM, 128), jnp.float32),
            pltpu.VMEM((_GRP, DIM, 128), jnp.float32),
            pltpu.VMEM((_B_PER_W * DIM,), jnp.float32),
            pltpu.SemaphoreType.DMA,
            pltpu.SemaphoreType.DMA,
        ],
        compiler_params=pltpu.CompilerParams(needs_layout_passes=False),
    )
    def gather_kernel(tbl, idx_hbm, out_hbm, idx_v, buf_a, buf_b, st, sa, sb):
        w = lax.axis_index("s") * _NUM_CORES + lax.axis_index("c")
        base_b = w * _B_PER_W
        pltpu.sync_copy(idx_hbm.at[pl.ds(base_b, _B_PER_W)], idx_v)

        iota = lax.iota(jnp.int32, 16)

        def enq(buf, slot, idx_scalar, sem):
            coff = pl.multiple_of((idx_scalar >> 7) * 128, 128)
            pltpu.async_copy(tbl.at[:, pl.ds(coff, 128)], buf.at[slot], sem)

        def drain(buf, slot, sem):
            pltpu.make_async_copy(
                tbl.at[:, pl.ds(0, 128)], buf.at[slot], sem
            ).wait()

        def extract(buf, slot, idx_scalar, j):
            lane = jnp.broadcast_to(idx_scalar & 127, (16,))
            g1 = plsc.load_gather(buf.at[slot], [iota, lane])
            g2 = plsc.load_gather(buf.at[slot], [iota + 16, lane])
            off = pl.multiple_of(j * DIM, 32)
            st[pl.ds(off, 16)] = g1
            st[pl.ds(off + 16, 16)] = g2

        vec0 = idx_v[pl.ds(0, 16)]
        for l in range(_GRP):
            enq(buf_a, l, vec0[l], sa)
        for l in range(_GRP):
            enq(buf_b, l, vec0[_GRP + l], sb)

        def body(i, vec_cur):
            nxt = jnp.minimum(i + 1, _NITER - 1) * 16
            vec_next = idx_v[pl.ds(pl.multiple_of(nxt, 16), 16)]

            for l in range(_GRP):
                drain(buf_a, l, sa)
            for l in range(_GRP):
                extract(buf_a, l, vec_cur[l], i * 16 + l)

            @pl.when(i < _NITER - 1)
            def _():
                for l in range(_GRP):
                    enq(buf_a, l, vec_next[l], sa)

            for l in range(_GRP):
                drain(buf_b, l, sb)
            for l in range(_GRP):
                extract(buf_b, l, vec_cur[_GRP + l], i * 16 + _GRP + l)

            @pl.when(i < _NITER - 1)
            def _():
                for l in range(_GRP):
                    enq(buf_b, l, vec_next[_GRP + l], sb)

            return vec_next

        lax.fori_loop(0, _NITER, body, vec0)
        pltpu.sync_copy(st, out_hbm.at[pl.ds(base_b * DIM, _B_PER_W * DIM)])

    return gather_kernel


_gather = _build_gather()


def kernel(table, feedid):
    out_flat = _gather(table.T, feedid)
    return out_flat.reshape(BATCH, DIM)
